# baseline (speedup 1.0000x reference)
import functools

import jax
import jax.numpy as jnp
from jax import lax
from jax.experimental import pallas as pl
from jax.experimental.pallas import tpu as pltpu

_BT = 128
_N_ROWS = 2816


def _gmm_body(poff_ref, cnt_ref, x_ref, w_ref, fc1_ref, fc2_ref, out_ref, *,
              bt, d_half):
    e = pl.program_id(0)
    start_e = poff_ref[e]
    nt = (cnt_ref[e] + bt - 1) // bt

    def body(i, _):
        start = pl.multiple_of(start_e + i * bt, 8)
        rows = x_ref[pl.ds(start, bt), :]
        y = lax.dot_general(rows, fc1_ref[0], (((1,), (1,)), ((), ())),
                            preferred_element_type=jnp.float32)
        y1 = y[:, :d_half]
        gate = y[:, d_half:]
        h = y1 * (gate * jax.nn.sigmoid(gate))
        yo = lax.dot_general(h, fc2_ref[0], (((1,), (1,)), ((), ())),
                             preferred_element_type=jnp.float32)
        out_ref[pl.ds(start, bt), :] = yo * w_ref[pl.ds(start, bt), :]
        return 0

    lax.fori_loop(0, nt, body, 0)


def _grouped_mlp(poff, counts, x_pad, w_pad, fc1_weights, fc2_weights, *,
                 bt=_BT):
    n_rows, d_model = x_pad.shape
    n_experts, d_ff2, _ = fc1_weights.shape
    d_half = d_ff2 // 2
    grid_spec = pltpu.PrefetchScalarGridSpec(
        num_scalar_prefetch=2,
        grid=(n_experts,),
        in_specs=[
            pl.BlockSpec((n_rows, d_model), lambda e, poff, cnt: (0, 0)),
            pl.BlockSpec((n_rows, 1), lambda e, poff, cnt: (0, 0)),
            pl.BlockSpec((1, d_ff2, d_model), lambda e, poff, cnt: (e, 0, 0)),
            pl.BlockSpec((1, d_model, d_half), lambda e, poff, cnt: (e, 0, 0)),
        ],
        out_specs=pl.BlockSpec((n_rows, d_model), lambda e, poff, cnt: (0, 0)),
    )
    return pl.pallas_call(
        functools.partial(_gmm_body, bt=bt, d_half=d_half),
        grid_spec=grid_spec,
        out_shape=jax.ShapeDtypeStruct((n_rows, d_model), jnp.float32),
    )(poff, counts, x_pad, w_pad, fc1_weights, fc2_weights)


def kernel(x, weights, indices, fc1_weights, fc2_weights):
    n_tokens = x.shape[0]
    n_experts = fc1_weights.shape[0]
    n_rows = _N_ROWS

    idx = indices[:, 0].astype(jnp.int32)
    sort_idx = jnp.argsort(idx)
    counts = jnp.zeros((n_experts,), jnp.int32).at[idx].add(1)
    off = jnp.concatenate(
        [jnp.zeros((1,), jnp.int32), jnp.cumsum(counts).astype(jnp.int32)])
    pcounts = (counts + 7) // 8 * 8
    poff = jnp.concatenate(
        [jnp.zeros((1,), jnp.int32), jnp.cumsum(pcounts).astype(jnp.int32)])

    idx_sorted = jnp.take(idx, sort_idx)
    pos_sorted = jnp.take(poff[:-1], idx_sorted) + (
        jnp.arange(n_tokens, dtype=jnp.int32) - jnp.take(off[:-1], idx_sorted))
    x_pad = jnp.zeros((n_rows, x.shape[1]), x.dtype).at[pos_sorted].set(
        jnp.take(x, sort_idx, axis=0))
    w_pad = jnp.zeros((n_rows, 1), jnp.float32).at[pos_sorted].set(
        jnp.take(weights[:, :1], sort_idx, axis=0))

    out_pad = _grouped_mlp(poff[:-1], counts, x_pad, w_pad,
                           fc1_weights, fc2_weights)

    pos = jnp.zeros((n_tokens,), jnp.int32).at[sort_idx].set(pos_sorted)
    return jnp.take(out_pad, pos, axis=0)


# speedup vs baseline: 2.0467x; 2.0467x over previous
import functools

import jax
import jax.numpy as jnp
from jax import lax
from jax.experimental import pallas as pl
from jax.experimental.pallas import tpu as pltpu
from jax.experimental.pallas import tpu_sc as plsc

_T = 2048
_E = 64
_D = 768
_BT = 128
_N_ROWS = 2816
_NW = 32
_TPW = _T // _NW

_MESH = dict(core_axis_name="c", subcore_axis_name="s")


def _route_body(idx_hbm, w_hbm, poff_hbm, cnt_hbm, pos_hbm, wpad_hbm,
                idx_v, w_v, pos_v, wpad_v, hist_v, sbase_v, cntr_v,
                poff_v, cnts_v):
    c = lax.axis_index("c")
    s = lax.axis_index("s")

    @pl.when((c == 0) & (s == 0))
    def _():
        pltpu.sync_copy(idx_hbm, idx_v)
        pltpu.sync_copy(w_hbm, w_v)
        lanes = lax.iota(jnp.int32, 16)
        ones = jnp.ones((16,), jnp.int32)
        zeros = jnp.zeros((16,), jnp.int32)

        def zbody(i, carry):
            hist_v[pl.ds(i * 16, 16)] = zeros
            cntr_v[pl.ds(i * 16, 16)] = zeros
            return carry

        lax.fori_loop(0, 64, zbody, 0)

        def p1(j, carry):
            e = plsc.load_gather(idx_v, [lanes * 128 + j])
            plsc.addupdate_scatter(hist_v, [lanes * 64 + e], ones)
            return carry

        lax.fori_loop(0, 128, p1, 0)

        for ch in range(4):
            run = zeros
            for l in range(16):
                sbase_v[pl.ds(l * 64 + ch * 16, 16)] = run
                run = run + hist_v[pl.ds(l * 64 + ch * 16, 16)]
            cnts_v[pl.ds(ch * 16, 16)] = run

        carry = zeros
        for ch in range(4):
            cnt = cnts_v[pl.ds(ch * 16, 16)]
            pc = ((cnt + 7) >> 3) << 3
            incl = plsc.cumsum(pc)
            poff_v[pl.ds(ch * 16, 16)] = incl - pc + carry
            carry = carry + jnp.broadcast_to(jnp.sum(pc), (16,))

        def p4(j, carry):
            tok = lanes * 128 + j
            e = plsc.load_gather(idx_v, [tok])
            stripe = lanes * 64 + e
            base = plsc.load_gather(poff_v, [e])
            sb = plsc.load_gather(sbase_v, [stripe])
            cr = plsc.load_gather(cntr_v, [stripe])
            p = base + sb + cr
            plsc.store_scatter(cntr_v, [stripe], cr + ones)
            plsc.store_scatter(pos_v, [tok], p)
            wv = plsc.load_gather(w_v, [tok])
            plsc.store_scatter(wpad_v, [p], wv)
            return carry

        lax.fori_loop(0, 128, p4, 0)

        pltpu.sync_copy(poff_v, poff_hbm)
        pltpu.sync_copy(cnts_v, cnt_hbm)
        pltpu.sync_copy(pos_v, pos_hbm)
        pltpu.sync_copy(wpad_v, wpad_hbm)


def _route(idx, w):
    return pl.kernel(
        _route_body,
        out_type=[
            jax.ShapeDtypeStruct((_E,), jnp.int32),
            jax.ShapeDtypeStruct((_E,), jnp.int32),
            jax.ShapeDtypeStruct((_T,), jnp.int32),
            jax.ShapeDtypeStruct((_N_ROWS,), jnp.float32),
        ],
        mesh=plsc.VectorSubcoreMesh(**_MESH),
        compiler_params=pltpu.CompilerParams(needs_layout_passes=False),
        scratch_types=[
            pltpu.VMEM((_T,), jnp.int32),
            pltpu.VMEM((_T,), jnp.float32),
            pltpu.VMEM((_T,), jnp.int32),
            pltpu.VMEM((_N_ROWS,), jnp.float32),
            pltpu.VMEM((1024,), jnp.int32),
            pltpu.VMEM((1024,), jnp.int32),
            pltpu.VMEM((1024,), jnp.int32),
            pltpu.VMEM((_E,), jnp.int32),
            pltpu.VMEM((_E,), jnp.int32),
        ],
    )(idx, w)


def _dispatch_body(x_hbm, pos_hbm, xpad_hbm, pos_v, rows_v, sem):
    wid = lax.axis_index("s") * 2 + lax.axis_index("c")
    base = wid * _TPW
    pltpu.sync_copy(pos_hbm.at[pl.ds(base, _TPW)], pos_v)
    pltpu.sync_copy(x_hbm.at[pl.ds(base, _TPW)], rows_v)
    pltpu.async_copy(rows_v, xpad_hbm.at[pos_v], sem).wait()


def _dispatch(x, pos):
    return pl.kernel(
        _dispatch_body,
        out_type=jax.ShapeDtypeStruct((_N_ROWS, _D), jnp.float32),
        mesh=plsc.VectorSubcoreMesh(**_MESH),
        scratch_types=[
            pltpu.VMEM((_TPW,), jnp.int32),
            pltpu.VMEM((_TPW, _D), jnp.float32),
            pltpu.SemaphoreType.DMA,
        ],
    )(x, pos)


def _gmm_body(poff_ref, cnt_ref, x_ref, w_ref, fc1_ref, fc2_ref, out_ref, *,
              bt, d_half):
    e = pl.program_id(0)
    start_e = poff_ref[e]
    nt = (cnt_ref[e] + bt - 1) // bt

    def body(i, carry):
        start = pl.multiple_of(start_e + i * bt, 8)
        rows = x_ref[pl.ds(start, bt), :]
        y = lax.dot_general(rows, fc1_ref[0], (((1,), (1,)), ((), ())),
                            preferred_element_type=jnp.float32)
        y1 = y[:, :d_half]
        gate = y[:, d_half:]
        h = y1 * (gate * jax.nn.sigmoid(gate))
        yo = lax.dot_general(h, fc2_ref[0], (((1,), (1,)), ((), ())),
                             preferred_element_type=jnp.float32)
        out_ref[pl.ds(start, bt), :] = yo * w_ref[pl.ds(start, bt), :]
        return carry

    lax.fori_loop(0, nt, body, 0)


def _grouped_mlp(poff, counts, x_pad, w_pad, fc1_weights, fc2_weights, *,
                 bt=_BT):
    n_rows, d_model = x_pad.shape
    n_experts, d_ff2, _ = fc1_weights.shape
    d_half = d_ff2 // 2
    grid_spec = pltpu.PrefetchScalarGridSpec(
        num_scalar_prefetch=2,
        grid=(n_experts,),
        in_specs=[
            pl.BlockSpec((n_rows, d_model), lambda e, poff, cnt: (0, 0)),
            pl.BlockSpec((n_rows, 1), lambda e, poff, cnt: (0, 0)),
            pl.BlockSpec((1, d_ff2, d_model), lambda e, poff, cnt: (e, 0, 0)),
            pl.BlockSpec((1, d_model, d_half), lambda e, poff, cnt: (e, 0, 0)),
        ],
        out_specs=pl.BlockSpec((n_rows, d_model), lambda e, poff, cnt: (0, 0)),
    )
    return pl.pallas_call(
        functools.partial(_gmm_body, bt=bt, d_half=d_half),
        grid_spec=grid_spec,
        out_shape=jax.ShapeDtypeStruct((n_rows, d_model), jnp.float32),
    )(poff, counts, x_pad, w_pad, fc1_weights, fc2_weights)


def _combine_body(outpad_hbm, pos_hbm, out_hbm, pos_v, rows_v, sem):
    wid = lax.axis_index("s") * 2 + lax.axis_index("c")
    base = wid * _TPW
    pltpu.sync_copy(pos_hbm.at[pl.ds(base, _TPW)], pos_v)
    pltpu.async_copy(outpad_hbm.at[pos_v], rows_v, sem).wait()
    pltpu.sync_copy(rows_v, out_hbm.at[pl.ds(base, _TPW)])


def _combine(out_pad, pos):
    return pl.kernel(
        _combine_body,
        out_type=jax.ShapeDtypeStruct((_T, _D), jnp.float32),
        mesh=plsc.VectorSubcoreMesh(**_MESH),
        scratch_types=[
            pltpu.VMEM((_TPW,), jnp.int32),
            pltpu.VMEM((_TPW, _D), jnp.float32),
            pltpu.SemaphoreType.DMA,
        ],
    )(out_pad, pos)


def kernel(x, weights, indices, fc1_weights, fc2_weights):
    idx = indices.reshape(-1).astype(jnp.int32)
    w = weights.reshape(-1)
    poff, counts, pos, w_pad = _route(idx, w)
    x_pad = _dispatch(x, pos)
    out_pad = _grouped_mlp(poff, counts, x_pad, w_pad.reshape(-1, 1),
                           fc1_weights, fc2_weights)
    return _combine(out_pad, pos)


# speedup vs baseline: 2.4657x; 1.2047x over previous
import functools

import jax
import jax.numpy as jnp
from jax import lax
from jax.experimental import pallas as pl
from jax.experimental.pallas import tpu as pltpu
from jax.experimental.pallas import tpu_sc as plsc

_T = 2048
_E = 64
_D = 768
_BT = 128
_N_ROWS = 2816
_NW = 32
_TPW = _T // _NW

_MESH = dict(core_axis_name="c", subcore_axis_name="s")


def _route_body(idx_hbm, w_hbm, poff_hbm, cnt_hbm, pos_hbm, wpad_hbm,
                idx_v, w_v, pos_v, wpad_v, hist_v, sbase_v, cntr_v,
                poff_v, cnts_v):
    c = lax.axis_index("c")
    s = lax.axis_index("s")

    @pl.when((c == 0) & (s == 0))
    def _():
        pltpu.sync_copy(idx_hbm, idx_v)
        pltpu.sync_copy(w_hbm, w_v)
        lanes = lax.iota(jnp.int32, 16)
        ones = jnp.ones((16,), jnp.int32)
        zeros = jnp.zeros((16,), jnp.int32)

        def zbody(i, carry):
            hist_v[pl.ds(i * 16, 16)] = zeros
            cntr_v[pl.ds(i * 16, 16)] = zeros
            return carry

        lax.fori_loop(0, 64, zbody, 0)

        def p1(j, carry):
            e = plsc.load_gather(idx_v, [lanes * 128 + j])
            plsc.addupdate_scatter(hist_v, [lanes * 64 + e], ones)
            return carry

        lax.fori_loop(0, 128, p1, 0)

        for ch in range(4):
            run = zeros
            for l in range(16):
                sbase_v[pl.ds(l * 64 + ch * 16, 16)] = run
                run = run + hist_v[pl.ds(l * 64 + ch * 16, 16)]
            cnts_v[pl.ds(ch * 16, 16)] = run

        carry = zeros
        for ch in range(4):
            cnt = cnts_v[pl.ds(ch * 16, 16)]
            pc = ((cnt + 7) >> 3) << 3
            incl = plsc.cumsum(pc)
            poff_v[pl.ds(ch * 16, 16)] = incl - pc + carry
            carry = carry + jnp.broadcast_to(jnp.sum(pc), (16,))

        def p4(j, carry):
            tok = lanes * 128 + j
            e = plsc.load_gather(idx_v, [tok])
            stripe = lanes * 64 + e
            base = plsc.load_gather(poff_v, [e])
            sb = plsc.load_gather(sbase_v, [stripe])
            cr = plsc.load_gather(cntr_v, [stripe])
            p = base + sb + cr
            plsc.store_scatter(cntr_v, [stripe], cr + ones)
            plsc.store_scatter(pos_v, [tok], p)
            wv = plsc.load_gather(w_v, [tok])
            plsc.store_scatter(wpad_v, [p], wv)
            return carry

        lax.fori_loop(0, 128, p4, 0)

        pltpu.sync_copy(poff_v, poff_hbm)
        pltpu.sync_copy(cnts_v, cnt_hbm)
        pltpu.sync_copy(pos_v, pos_hbm)
        pltpu.sync_copy(wpad_v, wpad_hbm)


def _route(idx, w):
    return pl.kernel(
        _route_body,
        out_type=[
            jax.ShapeDtypeStruct((_E,), jnp.int32),
            jax.ShapeDtypeStruct((_E,), jnp.int32),
            jax.ShapeDtypeStruct((_T,), jnp.int32),
            jax.ShapeDtypeStruct((_N_ROWS,), jnp.float32),
        ],
        mesh=plsc.VectorSubcoreMesh(**_MESH),
        compiler_params=pltpu.CompilerParams(needs_layout_passes=False),
        scratch_types=[
            pltpu.VMEM((_T,), jnp.int32),
            pltpu.VMEM((_T,), jnp.float32),
            pltpu.VMEM((_T,), jnp.int32),
            pltpu.VMEM((_N_ROWS,), jnp.float32),
            pltpu.VMEM((1024,), jnp.int32),
            pltpu.VMEM((1024,), jnp.int32),
            pltpu.VMEM((1024,), jnp.int32),
            pltpu.VMEM((_E,), jnp.int32),
            pltpu.VMEM((_E,), jnp.int32),
        ],
    )(idx, w)


def _dispatch_body(x_hbm, pos_hbm, xpad_hbm, pos_v, rows_v, sem):
    wid = lax.axis_index("s") * 2 + lax.axis_index("c")
    base = wid * _TPW
    pltpu.sync_copy(pos_hbm.at[pl.ds(base, _TPW)], pos_v)
    pltpu.sync_copy(x_hbm.at[pl.ds(base, _TPW)], rows_v)
    pltpu.async_copy(rows_v, xpad_hbm.at[pos_v], sem).wait()


def _dispatch(x, pos):
    return pl.kernel(
        _dispatch_body,
        out_type=jax.ShapeDtypeStruct((_N_ROWS, _D), jnp.float32),
        mesh=plsc.VectorSubcoreMesh(**_MESH),
        scratch_types=[
            pltpu.VMEM((_TPW,), jnp.int32),
            pltpu.VMEM((_TPW, _D), jnp.float32),
            pltpu.SemaphoreType.DMA,
        ],
    )(x, pos)


def _gmm_body(poff_ref, cnt_ref, x_ref, w_ref, fc1_ref, fc2_ref, out_ref, *,
              bt, d_half, epb):
    g = pl.program_id(0)
    for k in range(epb):
        e = g * epb + k
        start_e = poff_ref[e]
        nt = (cnt_ref[e] + bt - 1) // bt

        def body(i, carry, k=k, start_e=start_e):
            start = pl.multiple_of(start_e + i * bt, 8)
            rows = x_ref[pl.ds(start, bt), :]
            y = lax.dot_general(rows, fc1_ref[k], (((1,), (1,)), ((), ())),
                                preferred_element_type=jnp.float32)
            y1 = y[:, :d_half]
            gate = y[:, d_half:]
            h = y1 * (gate * jax.nn.sigmoid(gate))
            yo = lax.dot_general(h, fc2_ref[k], (((1,), (1,)), ((), ())),
                                 preferred_element_type=jnp.float32)
            out_ref[pl.ds(start, bt), :] = yo * w_ref[pl.ds(start, bt), :]
            return carry

        lax.fori_loop(0, nt, body, 0)


def _grouped_mlp(poff, counts, x_pad, w_pad, fc1_weights, fc2_weights, *,
                 bt=_BT, epb=4):
    n_rows, d_model = x_pad.shape
    n_experts, d_ff2, _ = fc1_weights.shape
    d_half = d_ff2 // 2
    grid_spec = pltpu.PrefetchScalarGridSpec(
        num_scalar_prefetch=2,
        grid=(n_experts // epb,),
        in_specs=[
            pl.BlockSpec((n_rows, d_model), lambda g, poff, cnt: (0, 0)),
            pl.BlockSpec((n_rows, 1), lambda g, poff, cnt: (0, 0)),
            pl.BlockSpec((epb, d_ff2, d_model), lambda g, poff, cnt: (g, 0, 0)),
            pl.BlockSpec((epb, d_model, d_half), lambda g, poff, cnt: (g, 0, 0)),
        ],
        out_specs=pl.BlockSpec((n_rows, d_model), lambda g, poff, cnt: (0, 0)),
    )
    return pl.pallas_call(
        functools.partial(_gmm_body, bt=bt, d_half=d_half, epb=epb),
        grid_spec=grid_spec,
        out_shape=jax.ShapeDtypeStruct((n_rows, d_model), jnp.float32),
    )(poff, counts, x_pad, w_pad, fc1_weights, fc2_weights)


def _combine_body(outpad_hbm, pos_hbm, out_hbm, pos_v, rows_v, sem):
    wid = lax.axis_index("s") * 2 + lax.axis_index("c")
    base = wid * _TPW
    pltpu.sync_copy(pos_hbm.at[pl.ds(base, _TPW)], pos_v)
    pltpu.async_copy(outpad_hbm.at[pos_v], rows_v, sem).wait()
    pltpu.sync_copy(rows_v, out_hbm.at[pl.ds(base, _TPW)])


def _combine(out_pad, pos):
    return pl.kernel(
        _combine_body,
        out_type=jax.ShapeDtypeStruct((_T, _D), jnp.float32),
        mesh=plsc.VectorSubcoreMesh(**_MESH),
        scratch_types=[
            pltpu.VMEM((_TPW,), jnp.int32),
            pltpu.VMEM((_TPW, _D), jnp.float32),
            pltpu.SemaphoreType.DMA,
        ],
    )(out_pad, pos)


def kernel(x, weights, indices, fc1_weights, fc2_weights):
    idx = indices.reshape(-1).astype(jnp.int32)
    w = weights.reshape(-1)
    poff, counts, pos, w_pad = _route(idx, w)
    x_pad = _dispatch(x, pos)
    out_pad = _grouped_mlp(poff, counts, x_pad, w_pad.reshape(-1, 1),
                           fc1_weights, fc2_weights)
    return _combine(out_pad, pos)
